# Initial kernel scaffold; baseline (speedup 1.0000x reference)
import jax
import jax.numpy as jnp
from jax.experimental import pallas as pl


def kernel(x, edge_index, edge_attr, batch, x_emb1, x_emb2, W, b, ee1, ee2, bn_g, bn_b, feat_w, feat_b, p1_w, p1_b, p2_w, p2_b):
    raise NotImplementedError("write your pallas kernel here")


# speedup vs baseline: 3.0649x; 3.0649x over previous
import functools

import jax
import jax.numpy as jnp
from jax import lax
from jax.experimental import pallas as pl
from jax.experimental.pallas import tpu as pltpu
from jax.experimental.pallas import tpu_sc as plsc

N = 10000
D = 128
G = 64
L = 5
FEAT = 256
E = 640000

CH = 128
NW = 32
NCH = 160
EW = NCH * CH
EPAD = EW * NW
NP = 10112
NROWS = NP // 16


GRP = 8
NG = NCH // GRP


def _sc_gather_scatter(dw):
  mesh = plsc.VectorSubcoreMesh(core_axis_name="c", subcore_axis_name="s")

  @functools.partial(
      pl.kernel,
      out_type=jax.ShapeDtypeStruct((2 * NP, dw), jnp.float32),
      mesh=mesh,
      scratch_types=[
          pltpu.VMEM((2 * GRP, CH), jnp.int32),
          pltpu.VMEM((2 * GRP, CH), jnp.int32),
          pltpu.VMEM((CH, dw), jnp.float32),
          pltpu.VMEM((CH, dw), jnp.float32),
          pltpu.VMEM_SHARED((NP, dw), jnp.float32),
          pltpu.SemaphoreType.DMA,
          pltpu.SemaphoreType.DMA,
          pltpu.SemaphoreType.DMA,
          pltpu.SemaphoreType.DMA,
      ],
      compiler_params=pltpu.CompilerParams(use_tc_tiling_on_sc=(dw == 128)),
  )
  def k(table, rc, zeros, out, ib0, ib1, rows0, rows1, accum,
        sem_i0, sem_i1, sem_r0, sem_r1):
    cid = lax.axis_index("c")
    sid = lax.axis_index("s")
    wid = cid * 16 + sid
    t0 = sid * NROWS
    ibase = wid * 2 * NCH

    pltpu.async_copy(rc.at[pl.ds(ibase, 2 * GRP)], ib0, sem_i0)
    pltpu.sync_copy(zeros.at[pl.ds(t0, NROWS)], accum.at[pl.ds(t0, NROWS)])
    plsc.subcore_barrier()
    pltpu.make_async_copy(rc.at[pl.ds(ibase, 2 * GRP)], ib0, sem_i0).wait()
    pltpu.async_copy(rc.at[pl.ds(ibase + 2 * GRP, 2 * GRP)], ib1, sem_i1)
    pltpu.async_copy(table.at[ib0.at[0]], rows0, sem_r0)
    pltpu.async_copy(table.at[ib0.at[2]], rows1, sem_r1)

    def group(g, ib_cur, ib_nxt, sem_i_cur, sem_i_nxt):
      nbase = ibase + (g + 1) * 2 * GRP
      for j in range(GRP):
        rows, semr = ((rows0, sem_r0) if j % 2 == 0 else (rows1, sem_r1))
        pltpu.make_async_copy(table.at[ib_cur.at[2 * j]], rows, semr).wait()
        pltpu.sync_copy(rows, accum.at[ib_cur.at[2 * j + 1]], add=True)
        if j < GRP - 2:
          pltpu.async_copy(table.at[ib_cur.at[2 * (j + 2)]], rows, semr)
        elif j == GRP - 2:
          @pl.when(g + 1 < NG)
          def _pf0():
            pltpu.make_async_copy(
                rc.at[pl.ds(nbase, 2 * GRP)], ib_nxt, sem_i_nxt).wait()
            pltpu.async_copy(table.at[ib_nxt.at[0]], rows, semr)
        else:
          @pl.when(g + 1 < NG)
          def _pf1():
            pltpu.async_copy(table.at[ib_nxt.at[2]], rows, semr)

      @pl.when(g + 2 < NG)
      def _ld():
        pltpu.async_copy(
            rc.at[pl.ds(nbase + 2 * GRP, 2 * GRP)], ib_cur, sem_i_cur)

    def step(i, carry):
      g = i * 2
      group(g, ib0, ib1, sem_i0, sem_i1)
      group(g + 1, ib1, ib0, sem_i1, sem_i0)
      return carry

    lax.fori_loop(0, NG // 2, step, 0)
    plsc.subcore_barrier()
    pltpu.sync_copy(accum.at[pl.ds(t0, NROWS)],
                    out.at[pl.ds(cid * NP + t0, NROWS)])

  return k


def _tc_prologue(x, emb1p, emb2p, w0):
  def body(x_ref, e1_ref, e2_ref, w0_ref, xh_ref):
    x0 = x_ref[:, 0:1]
    x1 = x_ref[:, 1:2]
    oh0 = (lax.broadcasted_iota(jnp.int32, (N, 128), 1) == x0).astype(
        jnp.float32)
    oh1 = (lax.broadcasted_iota(jnp.int32, (N, 8), 1) == x1).astype(
        jnp.float32)
    h = jnp.dot(oh0, e1_ref[...], preferred_element_type=jnp.float32)
    h = h + jnp.dot(oh1, e2_ref[...], preferred_element_type=jnp.float32)
    xh_ref[...] = jnp.dot(h, w0_ref[...], preferred_element_type=jnp.float32)

  return pl.pallas_call(
      body, out_shape=jax.ShapeDtypeStruct((N, D), jnp.float32),
  )(x, emb1p, emb2p, w0)


def _bn_relu(pt, pb, xh, st, sb, beff, g, bb):
  out = pt + pb + xh + (st + sb) + beff
  mu = jnp.mean(out, axis=0, keepdims=True)
  var = jnp.mean((out - mu) ** 2, axis=0, keepdims=True)
  hn = (out - mu) * lax.rsqrt(var + 1e-5) * g + bb
  return jnp.maximum(hn, 0.0)


def _tc_bn_next(pt, pb, xh, st, sb, beff, g, bb, wn):
  def body(pt_ref, pb_ref, xh_ref, st_ref, sb_ref, beff_ref, g_ref, bb_ref,
           wn_ref, o_ref):
    h = _bn_relu(pt_ref[...], pb_ref[...], xh_ref[...], st_ref[...],
                 sb_ref[...], beff_ref[...], g_ref[...], bb_ref[...])
    o_ref[...] = jnp.dot(h, wn_ref[...], preferred_element_type=jnp.float32)

  return pl.pallas_call(
      body, out_shape=jax.ShapeDtypeStruct((N, D), jnp.float32),
  )(pt, pb, xh, st, sb, beff, g, bb, wn)


def _tc_final(pt, pb, xh, st, sb, beff, g, bb, fw, fb, batch2, p1w, p1b,
              p2w, p2b):
  def body(pt_ref, pb_ref, xh_ref, st_ref, sb_ref, beff_ref, g_ref, bb_ref,
           fw_ref, fb_ref, bt_ref, p1w_ref, p1b_ref, p2w_ref, p2b_ref,
           pooled_ref, pred_ref):
    h = _bn_relu(pt_ref[...], pb_ref[...], xh_ref[...], st_ref[...],
                 sb_ref[...], beff_ref[...], g_ref[...], bb_ref[...])
    hf = jnp.dot(h, fw_ref[...], preferred_element_type=jnp.float32)
    hf = hf + fb_ref[...]
    oh = (bt_ref[...] == lax.broadcasted_iota(jnp.int32, (N, G), 1)).astype(
        jnp.float32)
    psum = lax.dot_general(oh, hf, (((0,), (0,)), ((), ())),
                           preferred_element_type=jnp.float32)
    cnt = lax.dot_general(oh, jnp.ones((N, 1), jnp.float32),
                          (((0,), (0,)), ((), ())),
                          preferred_element_type=jnp.float32)
    pooled = psum / jnp.maximum(cnt, 1.0)
    hid = jnp.maximum(
        jnp.dot(pooled, p1w_ref[...], preferred_element_type=jnp.float32)
        + p1b_ref[...], 0.0)
    pred = jnp.dot(hid, p2w_ref[...], preferred_element_type=jnp.float32)
    pred = pred + p2b_ref[...]
    pooled_ref[...] = pooled
    pred_ref[...] = pred

  return pl.pallas_call(
      body,
      out_shape=(jax.ShapeDtypeStruct((G, FEAT), jnp.float32),
                 jax.ShapeDtypeStruct((G, 2), jnp.float32)),
  )(pt, pb, xh, st, sb, beff, g, bb, fw, fb, batch2, p1w, p1b, p2w, p2b)


def kernel(x, edge_index, edge_attr, batch, x_emb1, x_emb2, W, b, ee1, ee2,
           bn_g, bn_b, feat_w, feat_b, p1_w, p1_b, p2_w, p2_b):
  f32 = jnp.float32
  row = edge_index[0]
  col = edge_index[1]
  code = edge_attr[:, 0] * 3 + edge_attr[:, 1]
  pad = EPAD - E
  rowp = jnp.concatenate([row, jnp.zeros((pad,), jnp.int32)])
  colp = jnp.concatenate([col, jnp.full((pad,), N, jnp.int32)])
  codep = jnp.concatenate([code, jnp.zeros((pad,), jnp.int32)])

  def rc_pack(gi, si):
    g2 = gi.reshape(NW, NCH, 1, CH)
    s2 = si.reshape(NW, NCH, 1, CH)
    return jnp.concatenate([g2, s2], axis=2).reshape(NW * NCH * 2, CH)

  rc_main = rc_pack(rowp, colp)
  rc_scal = rc_pack(codep, colp)

  t1 = jnp.transpose(ee1[:, :, 0])
  t2 = jnp.transpose(ee2[:, :, 0])
  eetab = jnp.pad((t1[:, None, :] + t2[None, :, :]).reshape(18, L),
                  ((0, 0), (0, 16 - L)))
  zeros128 = jnp.zeros((NP, D), f32)
  zeros16 = jnp.zeros((NP, 16), f32)

  emb1p = jnp.pad(x_emb1, ((0, 128 - x_emb1.shape[0]), (0, 0)))
  emb2p = jnp.pad(x_emb2, ((0, 8 - x_emb2.shape[0]), (0, 0)))

  sc16 = _sc_gather_scatter(16)
  sc128 = _sc_gather_scatter(D)

  sacc = sc16(eetab, rc_scal, zeros16)
  xh = _tc_prologue(x, emb1p, emb2p, W[0])

  ee_self = ee1[:, 4, 0] + ee2[:, 0, 0]
  pooled = pred = None
  for l in range(L):
    part = sc128(xh, rc_main, zeros128)
    pt, pb = part[0:N], part[NP:NP + N]
    st = sacc[0:N, l:l + 1]
    sb = sacc[NP:NP + N, l:l + 1]
    beff = (b[l] + ee_self[l]).reshape(1, D)
    g = bn_g[l].reshape(1, D)
    bb = bn_b[l].reshape(1, D)
    if l < L - 1:
      xh = _tc_bn_next(pt, pb, xh, st, sb, beff, g, bb, W[l + 1])
    else:
      pooled, pred = _tc_final(
          pt, pb, xh, st, sb, beff, g, bb, feat_w,
          feat_b.reshape(1, FEAT), batch.reshape(N, 1), p1_w,
          p1_b.reshape(1, FEAT // 2), p2_w, p2_b.reshape(1, 2))
  return (pooled, pred)


# speedup vs baseline: 4.4933x; 1.4660x over previous
import functools

import jax
import jax.numpy as jnp
from jax import lax
from jax.experimental import pallas as pl
from jax.experimental.pallas import tpu as pltpu
from jax.experimental.pallas import tpu_sc as plsc

N = 10000
D = 128
G = 64
L = 5
FEAT = 256
E = 640000

CH = 128
NW = 32
NCH = 160
EW = NCH * CH
EPAD = EW * NW
NP = 10112
NROWS = NP // 16


GRP = 8
NG = NCH // GRP
REP = 64


def _sc_gather_scatter(dw):
  mesh = plsc.VectorSubcoreMesh(core_axis_name="c", subcore_axis_name="s")

  @functools.partial(
      pl.kernel,
      out_type=jax.ShapeDtypeStruct((2 * NP, dw), jnp.float32),
      mesh=mesh,
      scratch_types=[
          pltpu.VMEM((2 * GRP, CH), jnp.int32),
          pltpu.VMEM((2 * GRP, CH), jnp.int32),
          pltpu.VMEM((CH, dw), jnp.float32),
          pltpu.VMEM((CH, dw), jnp.float32),
          pltpu.VMEM_SHARED((NP, dw), jnp.float32),
          pltpu.SemaphoreType.DMA,
          pltpu.SemaphoreType.DMA,
          pltpu.SemaphoreType.DMA,
          pltpu.SemaphoreType.DMA,
      ],
      compiler_params=pltpu.CompilerParams(use_tc_tiling_on_sc=(dw == 128)),
  )
  def k(table, rc, zeros, out, ib0, ib1, rows0, rows1, accum,
        sem_i0, sem_i1, sem_r0, sem_r1):
    cid = lax.axis_index("c")
    sid = lax.axis_index("s")
    wid = cid * 16 + sid
    t0 = sid * NROWS
    ibase = wid * 2 * NCH

    pltpu.async_copy(rc.at[pl.ds(ibase, 2 * GRP)], ib0, sem_i0)
    pltpu.sync_copy(zeros.at[pl.ds(t0, NROWS)], accum.at[pl.ds(t0, NROWS)])
    plsc.subcore_barrier()
    pltpu.make_async_copy(rc.at[pl.ds(ibase, 2 * GRP)], ib0, sem_i0).wait()
    pltpu.async_copy(rc.at[pl.ds(ibase + 2 * GRP, 2 * GRP)], ib1, sem_i1)
    pltpu.async_copy(table.at[ib0.at[0]], rows0, sem_r0)
    pltpu.async_copy(table.at[ib0.at[2]], rows1, sem_r1)

    def group(g, ib_cur, ib_nxt, sem_i_cur, sem_i_nxt):
      nbase = ibase + (g + 1) * 2 * GRP
      for j in range(GRP):
        rows, semr = ((rows0, sem_r0) if j % 2 == 0 else (rows1, sem_r1))
        pltpu.make_async_copy(table.at[ib_cur.at[2 * j]], rows, semr).wait()
        pltpu.sync_copy(rows, accum.at[ib_cur.at[2 * j + 1]], add=True)
        if j < GRP - 2:
          pltpu.async_copy(table.at[ib_cur.at[2 * (j + 2)]], rows, semr)
        elif j == GRP - 2:
          @pl.when(g + 1 < NG)
          def _pf0():
            pltpu.make_async_copy(
                rc.at[pl.ds(nbase, 2 * GRP)], ib_nxt, sem_i_nxt).wait()
            pltpu.async_copy(table.at[ib_nxt.at[0]], rows, semr)
        else:
          @pl.when(g + 1 < NG)
          def _pf1():
            pltpu.async_copy(table.at[ib_nxt.at[2]], rows, semr)

      @pl.when(g + 2 < NG)
      def _ld():
        pltpu.async_copy(
            rc.at[pl.ds(nbase + 2 * GRP, 2 * GRP)], ib_cur, sem_i_cur)

    def step(i, carry):
      g = i * 2
      group(g, ib0, ib1, sem_i0, sem_i1)
      group(g + 1, ib1, ib0, sem_i1, sem_i0)
      return carry

    lax.fori_loop(0, NG // 2, step, 0)
    plsc.subcore_barrier()
    pltpu.sync_copy(accum.at[pl.ds(t0, NROWS)],
                    out.at[pl.ds(cid * NP + t0, NROWS)])

  return k


def _tc_prologue(x, emb1p, emb2p, w0):
  def body(x_ref, e1_ref, e2_ref, w0_ref, xh_ref):
    x0 = x_ref[:, 0:1]
    x1 = x_ref[:, 1:2]
    oh0 = (lax.broadcasted_iota(jnp.int32, (N, 128), 1) == x0).astype(
        jnp.float32)
    oh1 = (lax.broadcasted_iota(jnp.int32, (N, 8), 1) == x1).astype(
        jnp.float32)
    h = jnp.dot(oh0, e1_ref[...], preferred_element_type=jnp.float32)
    h = h + jnp.dot(oh1, e2_ref[...], preferred_element_type=jnp.float32)
    xh_ref[...] = jnp.dot(h, w0_ref[...], preferred_element_type=jnp.float32)

  return pl.pallas_call(
      body, out_shape=jax.ShapeDtypeStruct((N, D), jnp.float32),
  )(x, emb1p, emb2p, w0)


def _bn_relu(pt, pb, xh, st, sb, beff, g, bb):
  out = pt + pb + xh + (st + sb) + beff
  mu = jnp.mean(out, axis=0, keepdims=True)
  var = jnp.mean((out - mu) ** 2, axis=0, keepdims=True)
  hn = (out - mu) * lax.rsqrt(var + 1e-5) * g + bb
  return jnp.maximum(hn, 0.0)


def _tc_bn_next(pt, pb, xh, st, sb, beff, g, bb, wn):
  def body(pt_ref, pb_ref, xh_ref, st_ref, sb_ref, beff_ref, g_ref, bb_ref,
           wn_ref, o_ref):
    h = _bn_relu(pt_ref[...], pb_ref[...], xh_ref[...], st_ref[...],
                 sb_ref[...], beff_ref[...], g_ref[...], bb_ref[...])
    o_ref[...] = jnp.dot(h, wn_ref[...], preferred_element_type=jnp.float32)

  return pl.pallas_call(
      body, out_shape=jax.ShapeDtypeStruct((N, D), jnp.float32),
  )(pt, pb, xh, st, sb, beff, g, bb, wn)


def _tc_final(pt, pb, xh, st, sb, beff, g, bb, fw, fb, batch2, p1w, p1b,
              p2w, p2b):
  def body(pt_ref, pb_ref, xh_ref, st_ref, sb_ref, beff_ref, g_ref, bb_ref,
           fw_ref, fb_ref, bt_ref, p1w_ref, p1b_ref, p2w_ref, p2b_ref,
           pooled_ref, pred_ref):
    h = _bn_relu(pt_ref[...], pb_ref[...], xh_ref[...], st_ref[...],
                 sb_ref[...], beff_ref[...], g_ref[...], bb_ref[...])
    hf = jnp.dot(h, fw_ref[...], preferred_element_type=jnp.float32)
    hf = hf + fb_ref[...]
    oh = (bt_ref[...] == lax.broadcasted_iota(jnp.int32, (N, G), 1)).astype(
        jnp.float32)
    psum = lax.dot_general(oh, hf, (((0,), (0,)), ((), ())),
                           preferred_element_type=jnp.float32)
    cnt = lax.dot_general(oh, jnp.ones((N, 1), jnp.float32),
                          (((0,), (0,)), ((), ())),
                          preferred_element_type=jnp.float32)
    pooled = psum / jnp.maximum(cnt, 1.0)
    hid = jnp.maximum(
        jnp.dot(pooled, p1w_ref[...], preferred_element_type=jnp.float32)
        + p1b_ref[...], 0.0)
    pred = jnp.dot(hid, p2w_ref[...], preferred_element_type=jnp.float32)
    pred = pred + p2b_ref[...]
    pooled_ref[...] = pooled
    pred_ref[...] = pred

  return pl.pallas_call(
      body,
      out_shape=(jax.ShapeDtypeStruct((G, FEAT), jnp.float32),
                 jax.ShapeDtypeStruct((G, 2), jnp.float32)),
  )(pt, pb, xh, st, sb, beff, g, bb, fw, fb, batch2, p1w, p1b, p2w, p2b)


def kernel(x, edge_index, edge_attr, batch, x_emb1, x_emb2, W, b, ee1, ee2,
           bn_g, bn_b, feat_w, feat_b, p1_w, p1_b, p2_w, p2_b):
  f32 = jnp.float32
  row = edge_index[0]
  col = edge_index[1]
  code = edge_attr[:, 0] * 3 + edge_attr[:, 1]
  pad = EPAD - E
  rowp = jnp.concatenate([row, jnp.zeros((pad,), jnp.int32)])
  padcol = N + (jnp.arange(pad, dtype=jnp.int32) % (NP - N))
  colp = jnp.concatenate([col, padcol])
  rep_off = 18 * (jnp.arange(EPAD, dtype=jnp.int32) % REP)
  codep = jnp.concatenate([code, jnp.zeros((pad,), jnp.int32)]) + rep_off

  def rc_pack(gi, si):
    g2 = gi.reshape(NW, NCH, 1, CH)
    s2 = si.reshape(NW, NCH, 1, CH)
    return jnp.concatenate([g2, s2], axis=2).reshape(NW * NCH * 2, CH)

  rc_main = rc_pack(rowp, colp)
  rc_scal = rc_pack(codep, colp)

  t1 = jnp.transpose(ee1[:, :, 0])
  t2 = jnp.transpose(ee2[:, :, 0])
  eetab = jnp.tile(
      jnp.pad((t1[:, None, :] + t2[None, :, :]).reshape(18, L),
              ((0, 0), (0, 16 - L))), (REP, 1))
  zeros128 = jnp.zeros((NP, D), f32)
  zeros16 = jnp.zeros((NP, 16), f32)

  emb1p = jnp.pad(x_emb1, ((0, 128 - x_emb1.shape[0]), (0, 0)))
  emb2p = jnp.pad(x_emb2, ((0, 8 - x_emb2.shape[0]), (0, 0)))

  sc16 = _sc_gather_scatter(16)
  sc128 = _sc_gather_scatter(D)

  sacc = sc16(eetab, rc_scal, zeros16)
  xh = _tc_prologue(x, emb1p, emb2p, W[0])

  ee_self = ee1[:, 4, 0] + ee2[:, 0, 0]
  pooled = pred = None
  for l in range(L):
    part = sc128(xh, rc_main, zeros128)
    pt, pb = part[0:N], part[NP:NP + N]
    st = sacc[0:N, l:l + 1]
    sb = sacc[NP:NP + N, l:l + 1]
    beff = (b[l] + ee_self[l]).reshape(1, D)
    g = bn_g[l].reshape(1, D)
    bb = bn_b[l].reshape(1, D)
    if l < L - 1:
      xh = _tc_bn_next(pt, pb, xh, st, sb, beff, g, bb, W[l + 1])
    else:
      pooled, pred = _tc_final(
          pt, pb, xh, st, sb, beff, g, bb, feat_w,
          feat_b.reshape(1, FEAT), batch.reshape(N, 1), p1_w,
          p1_b.reshape(1, FEAT // 2), p2_w, p2_b.reshape(1, 2))
  return (pooled, pred)
